# baseline (speedup 1.0000x reference)
import jax
import jax.numpy as jnp
from jax import lax
from jax.experimental import pallas as pl
from jax.experimental.pallas import tpu as pltpu
from jax.experimental.pallas import tpu_sc as plsc

NUM_ENTITY = 100000
NUM_LABEL = 1000
D = 64
B = 4096
NNEG = 50
NPAD = 64
NC = 2
NS = 16
NW = NC * NS
BPW = B // NW
L = 16
DC = D // L
NG = NPAD // L


NROW = 104
TOFF = 52


def _sc_body(head, label, tail, np_cat, ent, lab,
             pos_out, sq1_out, sq2_out,
             hidx, lidx, tidx, np_idx,
             h_rows, t_rows, l_rows, hl, lt,
             b0, b1, b2, b3,
             pos_part, sq1, sq2,
             sem_h, sem_t, sem_l, s0, s1, s2, s3):
    wid = lax.axis_index("s") * NC + lax.axis_index("c")
    base = wid * BPW

    pltpu.sync_copy(head.at[pl.ds(base, BPW)], hidx)
    pltpu.sync_copy(label.at[pl.ds(base, BPW)], lidx)
    pltpu.sync_copy(tail.at[pl.ds(base, BPW)], tidx)
    pltpu.sync_copy(np_cat.at[pl.ds(base, BPW)], np_idx)
    ch = pltpu.async_copy(ent.at[hidx], h_rows, sem_h)
    ct = pltpu.async_copy(ent.at[tidx], t_rows, sem_t)
    cl = pltpu.async_copy(lab.at[lidx], l_rows, sem_l)
    ch.wait()
    ct.wait()
    cl.wait()

    zv = jnp.zeros((L,), jnp.float32)

    def _pro(i, carry):
        acc = zv
        for c in range(DC):
            sl = pl.ds(L * c, L)
            hv = h_rows[i, sl]
            lv = l_rows[i, sl]
            tv = t_rows[i, sl]
            hlv = hv + lv
            hl[i, sl] = hlv
            lt[i, sl] = lv - tv
            dd = hlv - tv
            acc = acc + dd * dd
        pos_part[i, pl.ds(0, L)] = acc
        return carry

    lax.fori_loop(0, BPW, _pro, 0)

    def _issue(i, buf, sem):
        pltpu.async_copy(ent.at[np_idx.at[i]], buf, sem)

    def _wait(i, buf, sem):
        pltpu.make_async_copy(ent.at[np_idx.at[i]], buf, sem).wait()

    lane = jnp.arange(L, dtype=jnp.int32)
    bits = [((lane >> k) & 1) == 1 for k in range(4)]
    perms = [lane ^ (1 << k) for k in range(4)]

    def _combine(a, b, k):
        s1 = jnp.where(bits[k], b, a)
        s2 = jnp.where(bits[k], a, b)
        return s1 + jnp.take_along_axis(s2, perms[k], axis=0)

    def _tree(vs):
        k = 0
        while len(vs) > 1:
            vs = [_combine(vs[2 * m], vs[2 * m + 1], k)
                  for m in range(len(vs) // 2)]
            k += 1
        return vs[0]

    def _push(stack, v):
        k = 0
        while stack and stack[-1][0] == k:
            _, u = stack.pop()
            v = _combine(u, v, k)
            k += 1
        stack.append((k, v))

    def _rowp(j, buf, ltv, hlv):
        a1 = None
        a2 = None
        for c in range(DC):
            sl = pl.ds(L * c, L)
            hp = buf[j, sl]
            tp = buf[TOFF + j, sl]
            d1 = hp + ltv[c]
            d2 = hlv[c] - tp
            t1 = d1 * d1
            t2 = d2 * d2
            a1 = t1 if a1 is None else a1 + t1
            a2 = t2 if a2 is None else a2 + t2
        return a1, a2

    NFULL = NNEG // L
    NTAIL = NNEG - NFULL * L

    def _compute(i, buf):
        ltv = [lt[i, pl.ds(L * c, L)] for c in range(DC)]
        hlv = [hl[i, pl.ds(L * c, L)] for c in range(DC)]

        def _gl(g, carry):
            s1 = []
            s2 = []
            for jj in range(L):
                p1, p2 = _rowp(g * L + jj, buf, ltv, hlv)
                _push(s1, p1)
                _push(s2, p2)
            sq1[i, pl.ds(L * g, L)] = s1[0][1]
            sq2[i, pl.ds(L * g, L)] = s2[0][1]
            return carry

        lax.fori_loop(0, NFULL, _gl, 0)

        pa1, pa2 = _rowp(NFULL * L, buf, ltv, hlv)
        pb1, pb2 = _rowp(NFULL * L + 1, buf, ltv, hlv)
        u1 = _combine(pa1, pb1, 0)
        u2 = _combine(pa2, pb2, 0)
        for k in range(1, 4):
            u1 = _combine(u1, zv, k)
            u2 = _combine(u2, zv, k)
        sq1[i, pl.ds(NFULL * L, L)] = u1
        sq2[i, pl.ds(NFULL * L, L)] = u2

    bufs = (b0, b1, b2, b3)
    sems = (s0, s1, s2, s3)
    _issue(0, b0, s0)
    _issue(1, b1, s1)
    _issue(2, b2, s2)

    def _outer(i4, carry):
        i = i4 * 4
        for q in range(4):
            ii = i + q
            bq = bufs[q]
            sq = sems[q]
            bn = bufs[(q + 3) % 4]
            sn = sems[(q + 3) % 4]
            _wait(ii, bq, sq)

            @pl.when(ii + 3 < BPW)
            def _(ii=ii, bn=bn, sn=sn):
                _issue(ii + 3, bn, sn)

            _compute(ii, bq)
        return carry

    lax.fori_loop(0, BPW // 4, _outer, 0)

    pltpu.sync_copy(pos_part, pos_out.at[pl.ds(base, BPW)])
    pltpu.sync_copy(sq1, sq1_out.at[pl.ds(base, BPW)])
    pltpu.sync_copy(sq2, sq2_out.at[pl.ds(base, BPW)])


_sc_call = pl.kernel(
    _sc_body,
    out_type=(
        jax.ShapeDtypeStruct((B, L), jnp.float32),
        jax.ShapeDtypeStruct((B, NPAD), jnp.float32),
        jax.ShapeDtypeStruct((B, NPAD), jnp.float32),
    ),
    mesh=plsc.VectorSubcoreMesh(
        core_axis_name="c", subcore_axis_name="s",
        num_cores=NC, num_subcores=NS,
    ),
    compiler_params=pltpu.CompilerParams(
        needs_layout_passes=False, use_tc_tiling_on_sc=False),
    scratch_types=[
        pltpu.VMEM((BPW,), jnp.int32),
        pltpu.VMEM((BPW,), jnp.int32),
        pltpu.VMEM((BPW,), jnp.int32),
        pltpu.VMEM((BPW, NROW), jnp.int32),
        pltpu.VMEM((BPW, D), jnp.float32),
        pltpu.VMEM((BPW, D), jnp.float32),
        pltpu.VMEM((BPW, D), jnp.float32),
        pltpu.VMEM((BPW, D), jnp.float32),
        pltpu.VMEM((BPW, D), jnp.float32),
        pltpu.VMEM((NROW, D), jnp.float32),
        pltpu.VMEM((NROW, D), jnp.float32),
        pltpu.VMEM((NROW, D), jnp.float32),
        pltpu.VMEM((NROW, D), jnp.float32),
        pltpu.VMEM((BPW, L), jnp.float32),
        pltpu.VMEM((BPW, NPAD), jnp.float32),
        pltpu.VMEM((BPW, NPAD), jnp.float32),
        pltpu.SemaphoreType.DMA,
        pltpu.SemaphoreType.DMA,
        pltpu.SemaphoreType.DMA,
        pltpu.SemaphoreType.DMA,
        pltpu.SemaphoreType.DMA,
        pltpu.SemaphoreType.DMA,
        pltpu.SemaphoreType.DMA,
    ],
)


def _tc_body(gamma_ref, pos_ref, sq1_ref, sq2_ref, out_ref):
    g = gamma_ref[0, 0]
    pos_sq = jnp.sum(pos_ref[...], axis=1, keepdims=True)
    pos_d = jnp.sqrt(pos_sq)
    d1 = jnp.sqrt(sq1_ref[...])
    d2 = jnp.sqrt(sq2_ref[...])
    term = g + 2.0 * pos_d - d1 - d2
    col = lax.broadcasted_iota(jnp.int32, (B, NPAD), 1)
    v = jnp.where(col < NNEG, jnp.maximum(term, 0.0), 0.0)
    out_ref[0, 0] = jnp.sum(v)


_tc_call = pl.pallas_call(
    _tc_body,
    out_shape=jax.ShapeDtypeStruct((1, 1), jnp.float32),
    in_specs=[
        pl.BlockSpec(memory_space=pltpu.SMEM),
        pl.BlockSpec(memory_space=pltpu.VMEM),
        pl.BlockSpec(memory_space=pltpu.VMEM),
        pl.BlockSpec(memory_space=pltpu.VMEM),
    ],
    out_specs=pl.BlockSpec(memory_space=pltpu.SMEM),
)


def kernel(head, label, tail, head_p, tail_p, embed_entity, embed_label, gamma):
    z2 = jnp.zeros((B, TOFF - NNEG), jnp.int32)
    np_cat = jnp.concatenate([head_p, z2, tail_p, z2], axis=1)
    pos_part, sq1, sq2 = _sc_call(head, label, tail, np_cat,
                                  embed_entity, embed_label)
    out = _tc_call(gamma.reshape(1, 1), pos_part, sq1, sq2)
    return out[0, 0]


# speedup vs baseline: 1.0086x; 1.0086x over previous
import jax
import jax.numpy as jnp
from jax import lax
from jax.experimental import pallas as pl
from jax.experimental.pallas import tpu as pltpu
from jax.experimental.pallas import tpu_sc as plsc

NUM_ENTITY = 100000
NUM_LABEL = 1000
D = 64
B = 4096
NNEG = 50
NPAD = 64
NC = 2
NS = 16
NW = NC * NS
BPW = B // NW
L = 16
DC = D // L
NG = NPAD // L


NROW = 104
TOFF = 52


def _sc_body(head, label, tail, np_cat, ent, lab,
             pos_out, sq1_out, sq2_out,
             hidx, lidx, tidx, np_idx,
             h_rows, t_rows, l_rows, hl, lt,
             b0, b1,
             pos_part, sq1, sq2,
             sem_h, sem_t, sem_l, s0, s1):
    wid = lax.axis_index("s") * NC + lax.axis_index("c")
    base = wid * BPW

    pltpu.sync_copy(head.at[pl.ds(base, BPW)], hidx)
    pltpu.sync_copy(label.at[pl.ds(base, BPW)], lidx)
    pltpu.sync_copy(tail.at[pl.ds(base, BPW)], tidx)
    pltpu.sync_copy(np_cat.at[pl.ds(base, BPW)], np_idx)
    ch = pltpu.async_copy(ent.at[hidx], h_rows, sem_h)
    ct = pltpu.async_copy(ent.at[tidx], t_rows, sem_t)
    cl = pltpu.async_copy(lab.at[lidx], l_rows, sem_l)
    ch.wait()
    ct.wait()
    cl.wait()

    zv = jnp.zeros((L,), jnp.float32)

    def _pro(i, carry):
        acc = zv
        for c in range(DC):
            sl = pl.ds(L * c, L)
            hv = h_rows[i, sl]
            lv = l_rows[i, sl]
            tv = t_rows[i, sl]
            hlv = hv + lv
            hl[i, sl] = hlv
            lt[i, sl] = lv - tv
            dd = hlv - tv
            acc = acc + dd * dd
        pos_part[i, pl.ds(0, L)] = acc
        return carry

    lax.fori_loop(0, BPW, _pro, 0)

    def _issue(i, buf, sem):
        pltpu.async_copy(ent.at[np_idx.at[i]], buf, sem)

    def _wait(i, buf, sem):
        pltpu.make_async_copy(ent.at[np_idx.at[i]], buf, sem).wait()

    lane = jnp.arange(L, dtype=jnp.int32)
    bits = [((lane >> k) & 1) == 1 for k in range(4)]
    perms = [lane ^ (1 << k) for k in range(4)]

    def _combine(a, b, k):
        s1 = jnp.where(bits[k], b, a)
        s2 = jnp.where(bits[k], a, b)
        return s1 + jnp.take_along_axis(s2, perms[k], axis=0)

    def _tree(vs):
        k = 0
        while len(vs) > 1:
            vs = [_combine(vs[2 * m], vs[2 * m + 1], k)
                  for m in range(len(vs) // 2)]
            k += 1
        return vs[0]

    def _push(stack, v):
        k = 0
        while stack and stack[-1][0] == k:
            _, u = stack.pop()
            v = _combine(u, v, k)
            k += 1
        stack.append((k, v))

    def _rowp(j, buf, ltv, hlv):
        a1 = None
        a2 = None
        for c in range(DC):
            sl = pl.ds(L * c, L)
            hp = buf[j, sl]
            tp = buf[TOFF + j, sl]
            d1 = hp + ltv[c]
            d2 = hlv[c] - tp
            t1 = d1 * d1
            t2 = d2 * d2
            a1 = t1 if a1 is None else a1 + t1
            a2 = t2 if a2 is None else a2 + t2
        return a1, a2

    NFULL = NNEG // L
    NTAIL = NNEG - NFULL * L

    def _compute(i, buf):
        ltv = [lt[i, pl.ds(L * c, L)] for c in range(DC)]
        hlv = [hl[i, pl.ds(L * c, L)] for c in range(DC)]

        def _gl(g, carry):
            s1 = []
            s2 = []
            for jj in range(L):
                p1, p2 = _rowp(g * L + jj, buf, ltv, hlv)
                _push(s1, p1)
                _push(s2, p2)
            sq1[i, pl.ds(L * g, L)] = s1[0][1]
            sq2[i, pl.ds(L * g, L)] = s2[0][1]
            return carry

        lax.fori_loop(0, NFULL, _gl, 0)

        pa1, pa2 = _rowp(NFULL * L, buf, ltv, hlv)
        pb1, pb2 = _rowp(NFULL * L + 1, buf, ltv, hlv)
        u1 = _combine(pa1, pb1, 0)
        u2 = _combine(pa2, pb2, 0)
        for k in range(1, 4):
            u1 = _combine(u1, zv, k)
            u2 = _combine(u2, zv, k)
        sq1[i, pl.ds(NFULL * L, L)] = u1
        sq2[i, pl.ds(NFULL * L, L)] = u2

    _issue(0, b0, s0)

    def _outer(i2, carry):
        i = i2 * 2
        _wait(i, b0, s0)
        _issue(i + 1, b1, s1)
        _compute(i, b0)
        _wait(i + 1, b1, s1)

        @pl.when(i + 2 < BPW)
        def _():
            _issue(i + 2, b0, s0)

        _compute(i + 1, b1)
        return carry

    lax.fori_loop(0, BPW // 2, _outer, 0)

    pltpu.sync_copy(pos_part, pos_out.at[pl.ds(base, BPW)])
    pltpu.sync_copy(sq1, sq1_out.at[pl.ds(base, BPW)])
    pltpu.sync_copy(sq2, sq2_out.at[pl.ds(base, BPW)])


_sc_call = pl.kernel(
    _sc_body,
    out_type=(
        jax.ShapeDtypeStruct((B, L), jnp.float32),
        jax.ShapeDtypeStruct((B, NPAD), jnp.float32),
        jax.ShapeDtypeStruct((B, NPAD), jnp.float32),
    ),
    mesh=plsc.VectorSubcoreMesh(
        core_axis_name="c", subcore_axis_name="s",
        num_cores=NC, num_subcores=NS,
    ),
    compiler_params=pltpu.CompilerParams(
        needs_layout_passes=False, use_tc_tiling_on_sc=False),
    scratch_types=[
        pltpu.VMEM((BPW,), jnp.int32),
        pltpu.VMEM((BPW,), jnp.int32),
        pltpu.VMEM((BPW,), jnp.int32),
        pltpu.VMEM((BPW, NROW), jnp.int32),
        pltpu.VMEM((BPW, D), jnp.float32),
        pltpu.VMEM((BPW, D), jnp.float32),
        pltpu.VMEM((BPW, D), jnp.float32),
        pltpu.VMEM((BPW, D), jnp.float32),
        pltpu.VMEM((BPW, D), jnp.float32),
        pltpu.VMEM((NROW, D), jnp.float32),
        pltpu.VMEM((NROW, D), jnp.float32),
        pltpu.VMEM((BPW, L), jnp.float32),
        pltpu.VMEM((BPW, NPAD), jnp.float32),
        pltpu.VMEM((BPW, NPAD), jnp.float32),
        pltpu.SemaphoreType.DMA,
        pltpu.SemaphoreType.DMA,
        pltpu.SemaphoreType.DMA,
        pltpu.SemaphoreType.DMA,
        pltpu.SemaphoreType.DMA,
    ],
)


def _tc_body(gamma_ref, pos_ref, sq1_ref, sq2_ref, out_ref):
    g = gamma_ref[0, 0]
    pos_sq = jnp.sum(pos_ref[...], axis=1, keepdims=True)
    pos_d = jnp.sqrt(pos_sq)
    d1 = jnp.sqrt(sq1_ref[...])
    d2 = jnp.sqrt(sq2_ref[...])
    term = g + 2.0 * pos_d - d1 - d2
    col = lax.broadcasted_iota(jnp.int32, (B, NPAD), 1)
    v = jnp.where(col < NNEG, jnp.maximum(term, 0.0), 0.0)
    out_ref[0, 0] = jnp.sum(v)


_tc_call = pl.pallas_call(
    _tc_body,
    out_shape=jax.ShapeDtypeStruct((1, 1), jnp.float32),
    in_specs=[
        pl.BlockSpec(memory_space=pltpu.SMEM),
        pl.BlockSpec(memory_space=pltpu.VMEM),
        pl.BlockSpec(memory_space=pltpu.VMEM),
        pl.BlockSpec(memory_space=pltpu.VMEM),
    ],
    out_specs=pl.BlockSpec(memory_space=pltpu.SMEM),
)


def kernel(head, label, tail, head_p, tail_p, embed_entity, embed_label, gamma):
    z2 = jnp.zeros((B, TOFF - NNEG), jnp.int32)
    np_cat = jnp.concatenate([head_p, z2, tail_p, z2], axis=1)
    pos_part, sq1, sq2 = _sc_call(head, label, tail, np_cat,
                                  embed_entity, embed_label)
    out = _tc_call(gamma.reshape(1, 1), pos_part, sq1, sq2)
    return out[0, 0]


# speedup vs baseline: 2.6764x; 2.6535x over previous
import jax
import jax.numpy as jnp
from jax import lax
from jax.experimental import pallas as pl
from jax.experimental.pallas import tpu as pltpu
from jax.experimental.pallas import tpu_sc as plsc

NUM_ENTITY = 100000
NUM_LABEL = 1000
D = 64
B = 4096
NNEG = 50
NPAD = 64
NC = 2
NS = 16
NW = NC * NS
BPW = B // NW
L = 16
DC = D // L
NG = NPAD // L


def _sc_body(head, label, tail, head_p, tail_p, ent, lab,
             pos_out, sq1_out, sq2_out,
             hidx, lidx, tidx, hp_idx, tp_idx,
             h_rows, t_rows, l_rows, hl, lt,
             hp0, hp1, tp0, tp1,
             pos_part, sq1, sq2,
             sem_h, sem_t, sem_l, s0, s1, s2, s3):
    wid = lax.axis_index("s") * NC + lax.axis_index("c")
    base = wid * BPW

    pltpu.sync_copy(head.at[pl.ds(base, BPW)], hidx)
    pltpu.sync_copy(label.at[pl.ds(base, BPW)], lidx)
    pltpu.sync_copy(tail.at[pl.ds(base, BPW)], tidx)
    pltpu.sync_copy(head_p.at[pl.ds(base, BPW)], hp_idx)
    pltpu.sync_copy(tail_p.at[pl.ds(base, BPW)], tp_idx)
    ch = pltpu.async_copy(ent.at[hidx], h_rows, sem_h)
    ct = pltpu.async_copy(ent.at[tidx], t_rows, sem_t)
    cl = pltpu.async_copy(lab.at[lidx], l_rows, sem_l)
    ch.wait()
    ct.wait()
    cl.wait()

    zv = jnp.zeros((L,), jnp.float32)

    def _pro(i, carry):
        acc = zv
        for c in range(DC):
            sl = pl.ds(L * c, L)
            hv = h_rows[i, sl]
            lv = l_rows[i, sl]
            tv = t_rows[i, sl]
            hlv = hv + lv
            hl[i, sl] = hlv
            lt[i, sl] = lv - tv
            dd = hlv - tv
            acc = acc + dd * dd
        pos_part[i, pl.ds(0, L)] = acc
        return carry

    lax.fori_loop(0, BPW, _pro, 0)

    def _issue(i, hpb, tpb, sh, st):
        pltpu.async_copy(ent.at[hp_idx.at[i]], hpb, sh)
        pltpu.async_copy(ent.at[tp_idx.at[i]], tpb, st)

    def _wait(i, hpb, tpb, sh, st):
        pltpu.make_async_copy(ent.at[hp_idx.at[i]], hpb, sh).wait()
        pltpu.make_async_copy(ent.at[tp_idx.at[i]], tpb, st).wait()

    lane = jnp.arange(L, dtype=jnp.int32)
    bits = [((lane >> k) & 1) == 1 for k in range(4)]
    perms = [lane ^ (1 << k) for k in range(4)]

    def _combine(a, b, k):
        s1 = jnp.where(bits[k], b, a)
        s2 = jnp.where(bits[k], a, b)
        return s1 + jnp.take_along_axis(s2, perms[k], axis=0)

    def _tree(vs):
        k = 0
        while len(vs) > 1:
            vs = [_combine(vs[2 * m], vs[2 * m + 1], k)
                  for m in range(len(vs) // 2)]
            k += 1
        return vs[0]

    def _push(stack, v):
        k = 0
        while stack and stack[-1][0] == k:
            _, u = stack.pop()
            v = _combine(u, v, k)
            k += 1
        stack.append((k, v))

    def _rowp(j, hpb, tpb, ltv, hlv):
        a1 = None
        a2 = None
        for c in range(DC):
            sl = pl.ds(L * c, L)
            hp = hpb[j, sl]
            tp = tpb[j, sl]
            d1 = hp + ltv[c]
            d2 = hlv[c] - tp
            t1 = d1 * d1
            t2 = d2 * d2
            a1 = t1 if a1 is None else a1 + t1
            a2 = t2 if a2 is None else a2 + t2
        return a1, a2

    NFULL = NNEG // L
    NTAIL = NNEG - NFULL * L

    def _compute(i, hpb, tpb):
        ltv = [lt[i, pl.ds(L * c, L)] for c in range(DC)]
        hlv = [hl[i, pl.ds(L * c, L)] for c in range(DC)]

        def _gl(g, carry):
            s1 = []
            s2 = []
            for jj in range(L):
                p1, p2 = _rowp(g * L + jj, hpb, tpb, ltv, hlv)
                _push(s1, p1)
                _push(s2, p2)
            sq1[i, pl.ds(L * g, L)] = s1[0][1]
            sq2[i, pl.ds(L * g, L)] = s2[0][1]
            return carry

        lax.fori_loop(0, NFULL, _gl, 0)

        pa1, pa2 = _rowp(NFULL * L, hpb, tpb, ltv, hlv)
        pb1, pb2 = _rowp(NFULL * L + 1, hpb, tpb, ltv, hlv)
        u1 = _combine(pa1, pb1, 0)
        u2 = _combine(pa2, pb2, 0)
        for k in range(1, 4):
            u1 = _combine(u1, zv, k)
            u2 = _combine(u2, zv, k)
        sq1[i, pl.ds(NFULL * L, L)] = u1
        sq2[i, pl.ds(NFULL * L, L)] = u2

    _issue(0, hp0, tp0, s0, s2)

    def _outer(i2, carry):
        i = i2 * 2
        _issue(i + 1, hp1, tp1, s1, s3)
        _wait(i, hp0, tp0, s0, s2)
        _compute(i, hp0, tp0)

        @pl.when(i + 2 < BPW)
        def _():
            _issue(i + 2, hp0, tp0, s0, s2)

        _wait(i + 1, hp1, tp1, s1, s3)
        _compute(i + 1, hp1, tp1)
        return carry

    lax.fori_loop(0, BPW // 2, _outer, 0)

    pltpu.sync_copy(pos_part, pos_out.at[pl.ds(base, BPW)])
    pltpu.sync_copy(sq1, sq1_out.at[pl.ds(base, BPW)])
    pltpu.sync_copy(sq2, sq2_out.at[pl.ds(base, BPW)])


_sc_call = pl.kernel(
    _sc_body,
    out_type=(
        jax.ShapeDtypeStruct((B, L), jnp.float32),
        jax.ShapeDtypeStruct((B, NPAD), jnp.float32),
        jax.ShapeDtypeStruct((B, NPAD), jnp.float32),
    ),
    mesh=plsc.VectorSubcoreMesh(
        core_axis_name="c", subcore_axis_name="s",
        num_cores=NC, num_subcores=NS,
    ),
    compiler_params=pltpu.CompilerParams(
        needs_layout_passes=False, use_tc_tiling_on_sc=False),
    scratch_types=[
        pltpu.VMEM((BPW,), jnp.int32),
        pltpu.VMEM((BPW,), jnp.int32),
        pltpu.VMEM((BPW,), jnp.int32),
        pltpu.VMEM((BPW, NNEG), jnp.int32),
        pltpu.VMEM((BPW, NNEG), jnp.int32),
        pltpu.VMEM((BPW, D), jnp.float32),
        pltpu.VMEM((BPW, D), jnp.float32),
        pltpu.VMEM((BPW, D), jnp.float32),
        pltpu.VMEM((BPW, D), jnp.float32),
        pltpu.VMEM((BPW, D), jnp.float32),
        pltpu.VMEM((NNEG, D), jnp.float32),
        pltpu.VMEM((NNEG, D), jnp.float32),
        pltpu.VMEM((NNEG, D), jnp.float32),
        pltpu.VMEM((NNEG, D), jnp.float32),
        pltpu.VMEM((BPW, L), jnp.float32),
        pltpu.VMEM((BPW, NPAD), jnp.float32),
        pltpu.VMEM((BPW, NPAD), jnp.float32),
        pltpu.SemaphoreType.DMA,
        pltpu.SemaphoreType.DMA,
        pltpu.SemaphoreType.DMA,
        pltpu.SemaphoreType.DMA,
        pltpu.SemaphoreType.DMA,
        pltpu.SemaphoreType.DMA,
        pltpu.SemaphoreType.DMA,
    ],
)


def _tc_body(gamma_ref, pos_ref, sq1_ref, sq2_ref, out_ref):
    g = gamma_ref[0, 0]
    pos_sq = jnp.sum(pos_ref[...], axis=1, keepdims=True)
    pos_d = jnp.sqrt(pos_sq)
    d1 = jnp.sqrt(sq1_ref[...])
    d2 = jnp.sqrt(sq2_ref[...])
    term = g + 2.0 * pos_d - d1 - d2
    col = lax.broadcasted_iota(jnp.int32, (B, NPAD), 1)
    v = jnp.where(col < NNEG, jnp.maximum(term, 0.0), 0.0)
    out_ref[0, 0] = jnp.sum(v)


_tc_call = pl.pallas_call(
    _tc_body,
    out_shape=jax.ShapeDtypeStruct((1, 1), jnp.float32),
    in_specs=[
        pl.BlockSpec(memory_space=pltpu.SMEM),
        pl.BlockSpec(memory_space=pltpu.VMEM),
        pl.BlockSpec(memory_space=pltpu.VMEM),
        pl.BlockSpec(memory_space=pltpu.VMEM),
    ],
    out_specs=pl.BlockSpec(memory_space=pltpu.SMEM),
)


def kernel(head, label, tail, head_p, tail_p, embed_entity, embed_label, gamma):
    pos_part, sq1, sq2 = _sc_call(head, label, tail, head_p, tail_p,
                                  embed_entity, embed_label)
    out = _tc_call(gamma.reshape(1, 1), pos_part, sq1, sq2)
    return out[0, 0]
